# Initial kernel scaffold; baseline (speedup 1.0000x reference)
import jax
import jax.numpy as jnp
from jax.experimental import pallas as pl


def kernel(x, edge_index, batch, emb, W1, a_src1, a_dst1, b1, W2, a_src2, a_dst2, b2):
    raise NotImplementedError("write your pallas kernel here")


# speedup vs baseline: 1.0006x; 1.0006x over previous
import functools

import jax
import jax.numpy as jnp
from jax import lax
from jax.experimental import pallas as pl
from jax.experimental.pallas import tpu as pltpu

N = 100000
E = 3200000
NUM_FEAT = 128
HID = 16
NUM_CLASSES = 40
NUM_GRAPHS = 64

BN = 5000


def _emb_body(x_ref, emb_ref, h_ref):
    x = x_ref[...]
    pos = lax.broadcasted_iota(jnp.int32, x.shape, 1)
    rowmax = jnp.max(x, axis=1, keepdims=True)
    first = jnp.min(jnp.where(x == rowmax, pos, NUM_FEAT), axis=1, keepdims=True)
    onehot = (pos == first).astype(jnp.float32)
    h_ref[...] = jnp.dot(onehot, emb_ref[...], preferred_element_type=jnp.float32)


def _embed(x, emb):
    return pl.pallas_call(
        _emb_body,
        grid=(N // BN,),
        in_specs=[
            pl.BlockSpec((BN, NUM_FEAT), lambda i: (i, 0)),
            pl.BlockSpec((NUM_FEAT, HID), lambda i: (0, 0)),
        ],
        out_specs=pl.BlockSpec((BN, HID), lambda i: (i, 0)),
        out_shape=jax.ShapeDtypeStruct((N, HID), jnp.float32),
    )(x, emb)


def _gat_layer(h, W, a_src, a_dst, b, src, dst):
    hW = h @ W
    s = hW @ a_src
    d = hW @ a_dst
    e = jax.nn.leaky_relu(s[src] + d[dst], negative_slope=0.2)
    m = jax.ops.segment_max(e, dst, num_segments=N)
    m = jnp.where(jnp.isfinite(m), m, 0.0)
    ee = jnp.exp(e - m[dst])
    denom = jax.ops.segment_sum(ee, dst, num_segments=N)
    alpha = ee / jnp.maximum(denom[dst], 1e-16)
    out = jax.ops.segment_sum(alpha[:, None] * hW[src], dst, num_segments=N)
    return out + b


def kernel(x, edge_index, batch, emb, W1, a_src1, a_dst1, b1, W2, a_src2, a_dst2, b2):
    h = _embed(x, emb)
    loops = jnp.arange(N, dtype=edge_index.dtype)
    src = jnp.concatenate([edge_index[0], loops])
    dst = jnp.concatenate([edge_index[1], loops])
    h = _gat_layer(h, W1, a_src1, a_dst1, b1, src, dst)
    h = jax.nn.relu(h)
    h = _gat_layer(h, W2, a_src2, a_dst2, b2, src, dst)
    sums = jax.ops.segment_sum(h, batch, num_segments=NUM_GRAPHS)
    counts = jax.ops.segment_sum(jnp.ones((N, 1), dtype=h.dtype), batch, num_segments=NUM_GRAPHS)
    pooled = sums / jnp.maximum(counts, 1.0)
    return jax.nn.softmax(pooled, axis=1)


# speedup vs baseline: 20.5121x; 20.5004x over previous
import functools

import jax
import jax.numpy as jnp
from jax import lax
from jax.experimental import pallas as pl
from jax.experimental.pallas import tpu as pltpu
from jax.experimental.pallas import tpu_sc as plsc

N = 100000
E = 3200000
NUM_FEAT = 128
HID = 16
NUM_CLASSES = 40
NUM_GRAPHS = 64

NW = 32
CH_ROWS = 8
CH = CH_ROWS * 128
TILE_ROWS = 784
NCHUNK = TILE_ROWS // CH_ROWS
EP = NW * TILE_ROWS * 128
ROWS128 = EP // 128
NP = 100096
TSLICE = NP // 16

BN = 5000

_mesh = plsc.VectorSubcoreMesh(core_axis_name="c", subcore_axis_name="s")


def _emb_body(x_ref, emb_ref, h_ref):
    x = x_ref[...]
    pos = lax.broadcasted_iota(jnp.int32, x.shape, 1)
    rowmax = jnp.max(x, axis=1, keepdims=True)
    first = jnp.min(jnp.where(x == rowmax, pos, NUM_FEAT), axis=1, keepdims=True)
    onehot = (pos == first).astype(jnp.float32)
    h_ref[...] = jnp.dot(onehot, emb_ref[...], preferred_element_type=jnp.float32)


def _embed(x, emb):
    return pl.pallas_call(
        _emb_body,
        grid=(N // BN,),
        in_specs=[
            pl.BlockSpec((BN, NUM_FEAT), lambda i: (i, 0)),
            pl.BlockSpec((NUM_FEAT, HID), lambda i: (0, 0)),
        ],
        out_specs=pl.BlockSpec((BN, HID), lambda i: (i, 0)),
        out_shape=jax.ShapeDtypeStruct((N, HID), jnp.float32),
    )(x, emb)


@functools.partial(
    pl.kernel,
    out_type=(
        jax.ShapeDtypeStruct((EP,), jnp.float32),
        jax.ShapeDtypeStruct((2, NP, 8), jnp.float32),
    ),
    mesh=_mesh,
    compiler_params=pltpu.CompilerParams(needs_layout_passes=False, use_tc_tiling_on_sc=False),
    scratch_types=[
        pltpu.VMEM((CH_ROWS, 128), jnp.int32),
        pltpu.VMEM((CH_ROWS, 128), jnp.int32),
        pltpu.VMEM((CH,), jnp.float32),
        pltpu.VMEM((CH,), jnp.float32),
        pltpu.VMEM((CH,), jnp.float32),
        pltpu.VMEM((CH, 8), jnp.float32),
        pltpu.VMEM_SHARED((NP, 8), jnp.float32),
        pltpu.SemaphoreType.DMA,
    ],
)
def _attn_sc(src2d, dst2d, s_hbm, d_hbm, zeros8,
             ee_out, den_out,
             src_v, dst_v, s_f, d_f, ee_f, ee8, den_acc, sem):
    c = lax.axis_index("c")
    sid = lax.axis_index("s")
    wid = sid * 2 + c
    pltpu.sync_copy(zeros8.at[pl.ds(sid * TSLICE, TSLICE), :],
                    den_acc.at[pl.ds(sid * TSLICE, TSLICE), :])
    pltpu.sync_copy(zeros8.at[pl.ds(0, CH), :], ee8)
    plsc.subcore_barrier()

    row0 = wid * TILE_ROWS
    ebase0 = wid * (TILE_ROWS * 128)

    def chunk(i, carry):
        r = row0 + i * CH_ROWS
        eb = ebase0 + i * CH
        pltpu.sync_copy(src2d.at[pl.ds(r, CH_ROWS), :], src_v)
        pltpu.sync_copy(dst2d.at[pl.ds(r, CH_ROWS), :], dst_v)
        for j in range(CH_ROWS):
            pltpu.async_copy(s_hbm.at[src_v.at[j]],
                             s_f.at[pl.ds(j * 128, 128)], sem).wait()
        for j in range(CH_ROWS):
            pltpu.async_copy(d_hbm.at[dst_v.at[j]],
                             d_f.at[pl.ds(j * 128, 128)], sem).wait()

        def grp(g, carry2):
            sl = pl.ds(g * 16, 16)
            s16 = s_f[sl]
            d16 = d_f[sl]
            e16 = s16 + d16
            e16 = jnp.where(e16 >= 0.0, e16, e16 * jnp.float32(0.2))
            ee16 = jnp.exp(e16)
            ee_f[sl] = ee16
            eidx = g * 16 + lax.iota(jnp.int32, 16)
            plsc.store_scatter(ee8, [eidx, jnp.zeros((16,), jnp.int32)], ee16)
            return carry2

        lax.fori_loop(0, CH // 16, grp, 0)
        pltpu.sync_copy(ee_f, ee_out.at[pl.ds(eb, CH)])
        for j in range(CH_ROWS):
            pltpu.sync_copy(ee8.at[pl.ds(j * 128, 128), :],
                            den_acc.at[dst_v.at[j]], add=True)
        return carry

    lax.fori_loop(0, NCHUNK, chunk, 0)
    plsc.subcore_barrier()
    pltpu.sync_copy(den_acc.at[pl.ds(sid * TSLICE, TSLICE), :],
                    den_out.at[c, pl.ds(sid * TSLICE, TSLICE), :])


@functools.partial(
    pl.kernel,
    out_type=jax.ShapeDtypeStruct((2, NP, HID), jnp.float32),
    mesh=_mesh,
    compiler_params=pltpu.CompilerParams(needs_layout_passes=False, use_tc_tiling_on_sc=False),
    scratch_types=[
        pltpu.VMEM((CH_ROWS, 128), jnp.int32),
        pltpu.VMEM((CH_ROWS, 128), jnp.int32),
        pltpu.VMEM((CH,), jnp.float32),
        pltpu.VMEM((CH, HID), jnp.float32),
        pltpu.VMEM_SHARED((NP, HID), jnp.float32),
        pltpu.SemaphoreType.DMA,
    ],
)
def _feat_sc(src2d, dst2d, ee1d, hw_hbm, zeros16,
             acc_out,
             src_v, dst_v, ee_f, rows_v, acc, sem):
    c = lax.axis_index("c")
    sid = lax.axis_index("s")
    wid = sid * 2 + c
    pltpu.sync_copy(zeros16.at[pl.ds(sid * TSLICE, TSLICE), :],
                    acc.at[pl.ds(sid * TSLICE, TSLICE), :])
    plsc.subcore_barrier()

    row0 = wid * TILE_ROWS
    ebase0 = wid * (TILE_ROWS * 128)

    def chunk(i, carry):
        r = row0 + i * CH_ROWS
        eb = ebase0 + i * CH
        pltpu.sync_copy(src2d.at[pl.ds(r, CH_ROWS), :], src_v)
        pltpu.sync_copy(dst2d.at[pl.ds(r, CH_ROWS), :], dst_v)
        pltpu.sync_copy(ee1d.at[pl.ds(eb, CH)], ee_f)
        for j in range(CH_ROWS):
            pltpu.async_copy(hw_hbm.at[src_v.at[j]],
                             rows_v.at[pl.ds(j * 128, 128), :], sem).wait()

        def grp(g, carry2):
            sl = pl.ds(g * 16, 16)
            ee16 = ee_f[sl]
            eidx = g * 16 + lax.iota(jnp.int32, 16)
            for f in range(HID):
                fidx = jnp.full((16,), f, jnp.int32)
                vals = plsc.load_gather(rows_v, [eidx, fidx])
                plsc.store_scatter(rows_v, [eidx, fidx], vals * ee16)
            return carry2

        lax.fori_loop(0, CH // 16, grp, 0)
        for j in range(CH_ROWS):
            pltpu.sync_copy(rows_v.at[pl.ds(j * 128, 128), :],
                            acc.at[dst_v.at[j]], add=True)
        return carry

    lax.fori_loop(0, NCHUNK, chunk, 0)
    plsc.subcore_barrier()
    pltpu.sync_copy(acc.at[pl.ds(sid * TSLICE, TSLICE), :],
                    acc_out.at[c, pl.ds(sid * TSLICE, TSLICE), :])


def _gat_layer_sc(hW_list, s, d, b, src2d, dst2d, zeros8, zeros16):
    sp = jnp.concatenate([s, jnp.zeros((NP - N,), jnp.float32)])
    dp = jnp.concatenate([d, jnp.zeros((NP - N,), jnp.float32)])
    ee, den = _attn_sc(src2d, dst2d, sp, dp, zeros8)
    ee_self = jnp.exp(jax.nn.leaky_relu(s + d, negative_slope=0.2))
    denom = den[0, :N, 0] + den[1, :N, 0] + ee_self
    outs = []
    for hw in hW_list:
        acc = _feat_sc(src2d, dst2d, ee, hw, zeros16)
        outs.append(acc[0, :N, :] + acc[1, :N, :] + ee_self[:, None] * hw)
    out = jnp.concatenate(outs, axis=1) if len(outs) > 1 else outs[0]
    return out / denom[:, None] + b


def kernel(x, edge_index, batch, emb, W1, a_src1, a_dst1, b1, W2, a_src2, a_dst2, b2):
    f32 = jnp.float32
    h = _embed(x, emb)

    pad = EP - E
    src_p = jnp.concatenate([edge_index[0], jnp.zeros((pad,), jnp.int32)])
    dst_p = jnp.concatenate([edge_index[1], N + (jnp.arange(pad, dtype=jnp.int32) % (NP - N))])
    src2d = src_p.reshape(ROWS128, 128)
    dst2d = dst_p.reshape(ROWS128, 128)
    zeros8 = jnp.zeros((NP, 8), f32)
    zeros16 = jnp.zeros((NP, HID), f32)

    hW1 = h @ W1
    s1 = hW1 @ a_src1
    d1 = hW1 @ a_dst1
    out1 = _gat_layer_sc([hW1], s1, d1, b1, src2d, dst2d, zeros8, zeros16)
    out1 = jax.nn.relu(out1)

    hW2 = out1 @ W2
    s2 = hW2 @ a_src2
    d2 = hW2 @ a_dst2
    hw2p = jnp.concatenate([hW2, jnp.zeros((N, 48 - NUM_CLASSES), f32)], axis=1)
    chunks = [hw2p[:, i * 16:(i + 1) * 16] for i in range(3)]
    b2p = jnp.concatenate([b2, jnp.zeros((48 - NUM_CLASSES,), f32)])
    out2 = _gat_layer_sc(chunks, s2, d2, b2p, src2d, dst2d, zeros8, zeros16)
    out2 = out2[:, :NUM_CLASSES]

    sums = jax.ops.segment_sum(out2, batch, num_segments=NUM_GRAPHS)
    counts = jax.ops.segment_sum(jnp.ones((N, 1), f32), batch, num_segments=NUM_GRAPHS)
    pooled = sums / jnp.maximum(counts, 1.0)
    return jax.nn.softmax(pooled, axis=1)


# speedup vs baseline: 29.5815x; 1.4421x over previous
import functools

import jax
import jax.numpy as jnp
from jax import lax
from jax.experimental import pallas as pl
from jax.experimental.pallas import tpu as pltpu
from jax.experimental.pallas import tpu_sc as plsc

N = 100000
E = 3200000
NUM_FEAT = 128
HID = 16
NUM_CLASSES = 40
NUM_GRAPHS = 64

NW = 32
CH_ROWS = 8
CH = CH_ROWS * 128
TILE_ROWS = 784
NCHUNK = TILE_ROWS // CH_ROWS
EP = NW * TILE_ROWS * 128
ROWS128 = EP // 128
NP = 100096
TSLICE = NP // 16

BN = 5000

_mesh = plsc.VectorSubcoreMesh(core_axis_name="c", subcore_axis_name="s")


def _emb_body(x_ref, emb_ref, h_ref):
    x = x_ref[...]
    pos = lax.broadcasted_iota(jnp.int32, x.shape, 1)
    rowmax = jnp.max(x, axis=1, keepdims=True)
    first = jnp.min(jnp.where(x == rowmax, pos, NUM_FEAT), axis=1, keepdims=True)
    onehot = (pos == first).astype(jnp.float32)
    h_ref[...] = jnp.dot(onehot, emb_ref[...], preferred_element_type=jnp.float32)


def _embed(x, emb):
    return pl.pallas_call(
        _emb_body,
        grid=(N // BN,),
        in_specs=[
            pl.BlockSpec((BN, NUM_FEAT), lambda i: (i, 0)),
            pl.BlockSpec((NUM_FEAT, HID), lambda i: (0, 0)),
        ],
        out_specs=pl.BlockSpec((BN, HID), lambda i: (i, 0)),
        out_shape=jax.ShapeDtypeStruct((N, HID), jnp.float32),
    )(x, emb)


@functools.partial(
    pl.kernel,
    out_type=(
        jax.ShapeDtypeStruct((EP,), jnp.float32),
        jax.ShapeDtypeStruct((2, NP, 8), jnp.float32),
    ),
    mesh=_mesh,
    compiler_params=pltpu.CompilerParams(needs_layout_passes=False, use_tc_tiling_on_sc=False),
    scratch_types=[
        pltpu.VMEM((CH_ROWS, 128), jnp.int32),
        pltpu.VMEM((CH_ROWS, 128), jnp.int32),
        pltpu.VMEM((CH,), jnp.float32),
        pltpu.VMEM((CH,), jnp.float32),
        pltpu.VMEM((CH,), jnp.float32),
        pltpu.VMEM((CH, 8), jnp.float32),
        pltpu.VMEM_SHARED((NP, 8), jnp.float32),
        pltpu.SemaphoreType.DMA,
    ],
)
def _attn_sc(src2d, dst2d, s_hbm, d_hbm, zeros8,
             ee_out, den_out,
             src_v, dst_v, s_f, d_f, ee_f, ee8, den_acc, sem):
    c = lax.axis_index("c")
    sid = lax.axis_index("s")
    wid = sid * 2 + c
    pltpu.sync_copy(zeros8.at[pl.ds(sid * TSLICE, TSLICE), :],
                    den_acc.at[pl.ds(sid * TSLICE, TSLICE), :])
    pltpu.sync_copy(zeros8.at[pl.ds(0, CH), :], ee8)
    plsc.subcore_barrier()

    row0 = wid * TILE_ROWS
    ebase0 = wid * (TILE_ROWS * 128)

    def chunk(i, carry):
        r = row0 + i * CH_ROWS
        eb = ebase0 + i * CH
        pltpu.sync_copy(src2d.at[pl.ds(r, CH_ROWS), :], src_v)
        pltpu.sync_copy(dst2d.at[pl.ds(r, CH_ROWS), :], dst_v)
        hs = [pltpu.async_copy(s_hbm.at[src_v.at[j]],
                               s_f.at[pl.ds(j * 128, 128)], sem)
              for j in range(CH_ROWS)]
        hd = [pltpu.async_copy(d_hbm.at[dst_v.at[j]],
                               d_f.at[pl.ds(j * 128, 128)], sem)
              for j in range(CH_ROWS)]
        for h in hs:
            h.wait()
        for h in hd:
            h.wait()

        def grp(g, carry2):
            sl = pl.ds(g * 16, 16)
            s16 = s_f[sl]
            d16 = d_f[sl]
            e16 = s16 + d16
            e16 = jnp.where(e16 >= 0.0, e16, e16 * jnp.float32(0.2))
            ee16 = jnp.exp(e16)
            ee_f[sl] = ee16
            eidx = g * 16 + lax.iota(jnp.int32, 16)
            plsc.store_scatter(ee8, [eidx, jnp.zeros((16,), jnp.int32)], ee16)
            return carry2

        lax.fori_loop(0, CH // 16, grp, 0)
        pltpu.sync_copy(ee_f, ee_out.at[pl.ds(eb, CH)])
        hw = [pltpu.async_copy(ee8.at[pl.ds(j * 128, 128), :],
                               den_acc.at[dst_v.at[j]], sem, add=True)
              for j in range(CH_ROWS)]
        for h in hw:
            h.wait()
        return carry

    lax.fori_loop(0, NCHUNK, chunk, 0)
    plsc.subcore_barrier()
    pltpu.sync_copy(den_acc.at[pl.ds(sid * TSLICE, TSLICE), :],
                    den_out.at[c, pl.ds(sid * TSLICE, TSLICE), :])


@functools.partial(
    pl.kernel,
    out_type=jax.ShapeDtypeStruct((2, NP, HID), jnp.float32),
    mesh=_mesh,
    compiler_params=pltpu.CompilerParams(needs_layout_passes=False, use_tc_tiling_on_sc=False),
    scratch_types=[
        pltpu.VMEM((CH_ROWS, 128), jnp.int32),
        pltpu.VMEM((CH_ROWS, 128), jnp.int32),
        pltpu.VMEM((CH,), jnp.float32),
        pltpu.VMEM((CH, HID), jnp.float32),
        pltpu.VMEM_SHARED((NP, HID), jnp.float32),
        pltpu.SemaphoreType.DMA,
    ],
)
def _feat_sc(src2d, dst2d, ee1d, hw_hbm, zeros16,
             acc_out,
             src_v, dst_v, ee_f, rows_v, acc, sem):
    c = lax.axis_index("c")
    sid = lax.axis_index("s")
    wid = sid * 2 + c
    pltpu.sync_copy(zeros16.at[pl.ds(sid * TSLICE, TSLICE), :],
                    acc.at[pl.ds(sid * TSLICE, TSLICE), :])
    plsc.subcore_barrier()

    row0 = wid * TILE_ROWS
    ebase0 = wid * (TILE_ROWS * 128)

    def chunk(i, carry):
        r = row0 + i * CH_ROWS
        eb = ebase0 + i * CH
        pltpu.sync_copy(src2d.at[pl.ds(r, CH_ROWS), :], src_v)
        pltpu.sync_copy(dst2d.at[pl.ds(r, CH_ROWS), :], dst_v)
        pltpu.sync_copy(ee1d.at[pl.ds(eb, CH)], ee_f)
        hg = [pltpu.async_copy(hw_hbm.at[src_v.at[j]],
                               rows_v.at[pl.ds(j * 128, 128), :], sem)
              for j in range(CH_ROWS)]
        for h in hg:
            h.wait()

        def grp(g, carry2):
            sl = pl.ds(g * 16, 16)
            ee16 = ee_f[sl]
            eidx = g * 16 + lax.iota(jnp.int32, 16)
            for f in range(HID):
                fidx = jnp.full((16,), f, jnp.int32)
                vals = plsc.load_gather(rows_v, [eidx, fidx])
                plsc.store_scatter(rows_v, [eidx, fidx], vals * ee16)
            return carry2

        lax.fori_loop(0, CH // 16, grp, 0)
        hw = [pltpu.async_copy(rows_v.at[pl.ds(j * 128, 128), :],
                               acc.at[dst_v.at[j]], sem, add=True)
              for j in range(CH_ROWS)]
        for h in hw:
            h.wait()
        return carry

    lax.fori_loop(0, NCHUNK, chunk, 0)
    plsc.subcore_barrier()
    pltpu.sync_copy(acc.at[pl.ds(sid * TSLICE, TSLICE), :],
                    acc_out.at[c, pl.ds(sid * TSLICE, TSLICE), :])


def _gat_layer_sc(hW_list, s, d, b, src2d, dst2d, zeros8, zeros16):
    sp = jnp.concatenate([s, jnp.zeros((NP - N,), jnp.float32)])
    dp = jnp.concatenate([d, jnp.zeros((NP - N,), jnp.float32)])
    ee, den = _attn_sc(src2d, dst2d, sp, dp, zeros8)
    ee_self = jnp.exp(jax.nn.leaky_relu(s + d, negative_slope=0.2))
    denom = den[0, :N, 0] + den[1, :N, 0] + ee_self
    outs = []
    for hw in hW_list:
        acc = _feat_sc(src2d, dst2d, ee, hw, zeros16)
        outs.append(acc[0, :N, :] + acc[1, :N, :] + ee_self[:, None] * hw)
    out = jnp.concatenate(outs, axis=1) if len(outs) > 1 else outs[0]
    return out / denom[:, None] + b


def kernel(x, edge_index, batch, emb, W1, a_src1, a_dst1, b1, W2, a_src2, a_dst2, b2):
    f32 = jnp.float32
    h = _embed(x, emb)

    pad = EP - E
    src_p = jnp.concatenate([edge_index[0], jnp.zeros((pad,), jnp.int32)])
    dst_p = jnp.concatenate([edge_index[1], N + (jnp.arange(pad, dtype=jnp.int32) % (NP - N))])
    src2d = src_p.reshape(ROWS128, 128)
    dst2d = dst_p.reshape(ROWS128, 128)
    zeros8 = jnp.zeros((NP, 8), f32)
    zeros16 = jnp.zeros((NP, HID), f32)

    hW1 = h @ W1
    s1 = hW1 @ a_src1
    d1 = hW1 @ a_dst1
    out1 = _gat_layer_sc([hW1], s1, d1, b1, src2d, dst2d, zeros8, zeros16)
    out1 = jax.nn.relu(out1)

    hW2 = out1 @ W2
    s2 = hW2 @ a_src2
    d2 = hW2 @ a_dst2
    hw2p = jnp.concatenate([hW2, jnp.zeros((N, 48 - NUM_CLASSES), f32)], axis=1)
    chunks = [hw2p[:, i * 16:(i + 1) * 16] for i in range(3)]
    b2p = jnp.concatenate([b2, jnp.zeros((48 - NUM_CLASSES,), f32)])
    out2 = _gat_layer_sc(chunks, s2, d2, b2p, src2d, dst2d, zeros8, zeros16)
    out2 = out2[:, :NUM_CLASSES]

    sums = jax.ops.segment_sum(out2, batch, num_segments=NUM_GRAPHS)
    counts = jax.ops.segment_sum(jnp.ones((N, 1), f32), batch, num_segments=NUM_GRAPHS)
    pooled = sums / jnp.maximum(counts, 1.0)
    return jax.nn.softmax(pooled, axis=1)
